# Initial kernel scaffold; baseline (speedup 1.0000x reference)
import jax
import jax.numpy as jnp
from jax.experimental import pallas as pl


def kernel(x, t, weight):
    raise NotImplementedError("write your pallas kernel here")


# speedup vs baseline: 2.4476x; 2.4476x over previous
import functools

import jax
import jax.numpy as jnp
from jax.experimental import pallas as pl
from jax.experimental.pallas import tpu as pltpu

_CHARB_EPS2 = 1.0e-6


def _stencil_kernel(w_ref, x_ref, t_ref, o_ref, *, ksize, cpb, m):
    _, H, W = x_ref.shape
    Ho = H - ksize + 1
    Wo = W - ksize + 1

    def chan_body(c, tot):
        d = x_ref[c] - t_ref[c]
        win = [d[ki:ki + Ho, kj:kj + Wo]
               for ki in range(ksize) for kj in range(ksize)]
        for r in range(m):
            row = c * m + r
            acc = w_ref[row, 0] * win[0]
            for tap in range(1, ksize * ksize):
                acc = acc + w_ref[row, tap] * win[tap]
            tot = tot + jnp.sqrt(acc * acc + _CHARB_EPS2)
        return tot

    tot = jax.lax.fori_loop(0, cpb, chan_body,
                            jnp.zeros((Ho, Wo), jnp.float32))
    o_ref[...] = jnp.sum(tot, axis=0, keepdims=True)[None]


def kernel(x, t, weight):
    B, C, H, W = x.shape
    OC, _, K, _ = weight.shape
    m = OC // C
    Ho, Wo = H - K + 1, W - K + 1

    x3 = x.reshape(B * C, H, W).astype(jnp.float32)
    t3 = t.reshape(B * C, H, W).astype(jnp.float32)
    w2 = weight[:, 0].astype(jnp.float32).reshape(OC, K * K)

    out = pl.pallas_call(
        functools.partial(_stencil_kernel, ksize=K, cpb=C, m=m),
        grid=(B,),
        in_specs=[
            pl.BlockSpec(memory_space=pltpu.SMEM),
            pl.BlockSpec((C, H, W), lambda b: (b, 0, 0)),
            pl.BlockSpec((C, H, W), lambda b: (b, 0, 0)),
        ],
        out_specs=pl.BlockSpec((1, 1, Wo), lambda b: (b, 0, 0)),
        out_shape=jax.ShapeDtypeStruct((B, 1, Wo), jnp.float32),
        compiler_params=pltpu.CompilerParams(
            dimension_semantics=("parallel",),
        ),
    )(w2, x3, t3)

    denom = float(B * OC * Ho * Wo)
    return jnp.sum(out) / jnp.float32(denom)
